# baseline (speedup 1.0000x reference)
import functools

import jax
import jax.numpy as jnp
from jax import lax
from jax.experimental import pallas as pl
from jax.experimental.pallas import tpu as pltpu
from jax.experimental.pallas import tpu_sc as plsc

N = 10000
KNN = 16
NPAD = 10240
SLAB = 128
NSLAB = NPAD // SLAB
SBITS = 7
SMASK = (1 << SBITS) - 1
DEPTH = 4
RB_KNN = 80
RB_CONV = 400
RB_MLP = 2000


def _knn_body(prows_ref, pcols_ref, out_ref):
    i = pl.program_id(0)
    xr = prows_ref[...]
    xc = pcols_ref[...]
    sq_r = jnp.sum(xr * xr, axis=1, keepdims=True)
    sq_c = jnp.sum(xc * xc, axis=0, keepdims=True)
    prod = jnp.dot(xr, xc, preferred_element_type=jnp.float32)
    d = sq_r + sq_c - 2.0 * prod
    inf = jnp.float32(jnp.inf)
    big = jnp.int32(NPAD)
    bias = jnp.int32(0x08000000)
    sent = jnp.int32(0x7F000000)
    sentf = jnp.float32(2.0 ** 127)
    row_ids = i * RB_KNN + lax.broadcasted_iota(jnp.int32, (RB_KNN, 1), 0)
    lane = lax.broadcasted_iota(jnp.int32, (RB_KNN, SLAB), 1)

    def _ce(a, b):
        return jnp.minimum(a, b), jnp.maximum(a, b)

    K = [jnp.full((RB_KNN, SLAB), sent) for _ in range(DEPTH)]
    grp = []
    for c in range(NSLAB):
        v = jnp.maximum(d[:, c * SLAB:(c + 1) * SLAB], 0.0)
        bits = lax.bitcast_convert_type(v, jnp.int32)
        vk = ((bits & jnp.int32(~SMASK)) | jnp.int32(c)) + bias
        if (c + 1) * SLAB > N:
            nv = max(0, N - c * SLAB)
            vk = jnp.where(lane >= jnp.int32(nv), sent, vk)
        grp.append(vk)
        if len(grp) == 4:
            a0, a1 = _ce(grp[0], grp[1])
            b0, b1 = _ce(grp[2], grp[3])
            g0, t1 = _ce(a0, b0)
            t2, g3 = _ce(a1, b1)
            g1, g2 = _ce(t1, t2)
            g = [g0, g1, g2, g3]
            L = [jnp.minimum(K[k], g[3 - k]) for k in range(4)]
            p0, p2 = _ce(L[0], L[2])
            p1, p3 = _ce(L[1], L[3])
            K0, K1 = _ce(p0, p1)
            K2, K3 = _ce(p2, p3)
            K = [K0, K1, K2, K3]
            grp = []

    F = [lax.bitcast_convert_type(k, jnp.float32) for k in K]
    cols = []
    qb15 = qb16 = None
    for t in range(KNN + 1):
        m = jnp.min(F[0], axis=1, keepdims=True)
        sel_lane = jnp.min(jnp.where(F[0] == m, lane, jnp.int32(SLAB)),
                           axis=1, keepdims=True)
        islane = lane == sel_lane
        mi = lax.bitcast_convert_type(m, jnp.int32) - bias
        cols.append((mi & jnp.int32(SMASK)) * SLAB + sel_lane)
        for k in range(DEPTH - 1):
            F[k] = jnp.where(islane, F[k + 1], F[k])
        F[DEPTH - 1] = jnp.where(islane, sentf, F[DEPTH - 1])
        if t == KNN - 1:
            qb15 = mi >> SBITS
        if t == KNN:
            qb16 = mi >> SBITS
    k18 = jnp.min(F[0], axis=1, keepdims=True)
    qb17 = (lax.bitcast_convert_type(k18, jnp.int32) - bias) >> SBITS
    drained = jnp.max(F[0], axis=1, keepdims=True) == sentf

    run = jnp.zeros((RB_KNN, 1), jnp.bool_)
    for t in range(KNN):
        run = run | (cols[t] == row_ids)
        out_ref[:, t:t + 1] = jnp.where(run, cols[t + 1], cols[t])
    eq16 = cols[KNN] == row_ids

    one = jnp.int32(1)
    zero = jnp.int32(0)
    c_a = jnp.where(qb17 <= qb16, one, zero)
    c_b = jnp.where(qb17 <= qb15, one, zero)
    c_c = jnp.where(qb16 <= qb15, one, zero)
    c_d = jnp.where(drained, one, zero)
    bad_i = jnp.maximum(jnp.where(run, c_a, jnp.where(eq16, c_b, c_c)), c_d)
    bad = jnp.max(bad_i) > 0

    @pl.when(bad)
    def _slow():
        col = lax.broadcasted_iota(jnp.int32, (RB_KNN, NPAD), 1)
        dd = jnp.where((col == row_ids) | (col >= N), inf, d)
        for t in range(KNN):
            mm = jnp.min(dd, axis=1, keepdims=True)
            cand = jnp.where(dd == mm, col, big)
            sel = jnp.min(cand, axis=1, keepdims=True)
            out_ref[:, t:t + 1] = sel
            dd = jnp.where(col == sel, inf, dd)


def _knn(pts, interpret=False):
    prows = jnp.pad(pts, ((0, 0), (0, 5)))
    pcols = jnp.pad(pts.T, ((0, 5), (0, NPAD - N)))
    return pl.pallas_call(
        _knn_body,
        grid=(N // RB_KNN,),
        in_specs=[
            pl.BlockSpec((RB_KNN, 8), lambda i: (i, 0)),
            pl.BlockSpec((8, NPAD), lambda i: (0, 0)),
        ],
        out_specs=pl.BlockSpec((RB_KNN, KNN), lambda i: (i, 0)),
        out_shape=jax.ShapeDtypeStruct((N, KNN), jnp.int32),
        interpret=interpret,
    )(prows, pcols)


def _sc_gather(table, idx_flat):
    E = idx_flat.shape[0]
    C = table.shape[1]
    info = plsc.get_sparse_core_info()
    nw = info.num_cores * info.num_subcores
    per_w = E // nw
    nch = per_w // 128
    rem = per_w - nch * 128
    mesh = plsc.VectorSubcoreMesh(core_axis_name="c", subcore_axis_name="s")

    @functools.partial(
        pl.kernel,
        mesh=mesh,
        compiler_params=pltpu.CompilerParams(use_tc_tiling_on_sc=False),
        out_type=jax.ShapeDtypeStruct((E, C), jnp.float32),
        scratch_types=[
            pltpu.VMEM((128,), jnp.int32),
            pltpu.VMEM((128, C), jnp.float32),
            pltpu.VMEM((8,), jnp.int32),
            pltpu.VMEM((8, C), jnp.float32),
            pltpu.SemaphoreType.DMA,
        ],
    )
    def gk(table_hbm, idx_hbm, out_hbm, idx_v, rows_v, idx_v2, rows_v2, sem):
        wid = lax.axis_index("s") * info.num_cores + lax.axis_index("c")
        base = wid * per_w

        def body(ci, carry):
            off = base + ci * 128
            pltpu.sync_copy(idx_hbm.at[pl.ds(off, 128)], idx_v)
            pltpu.async_copy(table_hbm.at[idx_v], rows_v, sem).wait()
            pltpu.sync_copy(rows_v, out_hbm.at[pl.ds(off, 128)])
            return carry

        lax.fori_loop(0, nch, body, 0)
        if rem:
            off2 = base + nch * 128
            pltpu.sync_copy(idx_hbm.at[pl.ds(off2, rem)], idx_v2)
            pltpu.async_copy(table_hbm.at[idx_v2], rows_v2, sem).wait()
            pltpu.sync_copy(rows_v2, out_hbm.at[pl.ds(off2, rem)])

    return gk(table, idx_flat)


def _conv_body(x_ref, xj_ref, wc_ref, wn_ref, b1_ref, w2_ref, b2_ref, out_ref):
    x = x_ref[...]
    tcv = jnp.dot(x, wc_ref[...], preferred_element_type=jnp.float32) \
        + b1_ref[...]
    acc = None
    for j in range(KNN):
        xj = xj_ref[j]
        pre = jnp.dot(xj, wn_ref[...], preferred_element_type=jnp.float32) + tcv
        hj = jnp.maximum(pre, 0.0)
        oj = jnp.dot(hj, w2_ref[...], preferred_element_type=jnp.float32)
        acc = oj if acc is None else jnp.maximum(acc, oj)
    out_ref[...] = acc + b2_ref[...]


def _edge_conv(x, xj3, wc, wn, b1, w2, b2, interpret=False):
    C = x.shape[1]
    return pl.pallas_call(
        _conv_body,
        grid=(N // RB_CONV,),
        in_specs=[
            pl.BlockSpec((RB_CONV, C), lambda i: (i, 0)),
            pl.BlockSpec((KNN, RB_CONV, C), lambda i: (0, i, 0)),
            pl.BlockSpec((C, 64), lambda i: (0, 0)),
            pl.BlockSpec((C, 64), lambda i: (0, 0)),
            pl.BlockSpec((1, 64), lambda i: (0, 0)),
            pl.BlockSpec((64, 64), lambda i: (0, 0)),
            pl.BlockSpec((1, 64), lambda i: (0, 0)),
        ],
        out_specs=pl.BlockSpec((RB_CONV, 64), lambda i: (i, 0)),
        out_shape=jax.ShapeDtypeStruct((N, 64), jnp.float32),
        interpret=interpret,
    )(x, xj3, wc, wn, b1, w2, b2)


def _mlp_body(f1_ref, f2_ref, f3_ref, wa_ref, wb_ref, wc_ref, b7_ref,
              w8_ref, b8_ref, out_ref):
    t = (jnp.dot(f1_ref[...], wa_ref[...], preferred_element_type=jnp.float32)
         + jnp.dot(f2_ref[...], wb_ref[...], preferred_element_type=jnp.float32)
         + jnp.dot(f3_ref[...], wc_ref[...], preferred_element_type=jnp.float32)
         + b7_ref[...])
    t = jnp.maximum(t, 0.0)
    out_ref[...] = jnp.dot(t, w8_ref[...], preferred_element_type=jnp.float32) \
        + b8_ref[...]


def _final_mlp(f1, f2, f3, w7, b7, w8, b8, interpret=False):
    w8p = jnp.pad(w8, ((0, 0), (0, 8 - w8.shape[1])))
    b8p = jnp.pad(b8, (0, 8 - b8.shape[0]))[None, :]
    out = pl.pallas_call(
        _mlp_body,
        grid=(N // RB_MLP,),
        in_specs=[
            pl.BlockSpec((RB_MLP, 64), lambda i: (i, 0)),
            pl.BlockSpec((RB_MLP, 64), lambda i: (i, 0)),
            pl.BlockSpec((RB_MLP, 64), lambda i: (i, 0)),
            pl.BlockSpec((64, 256), lambda i: (0, 0)),
            pl.BlockSpec((64, 256), lambda i: (0, 0)),
            pl.BlockSpec((64, 256), lambda i: (0, 0)),
            pl.BlockSpec((1, 256), lambda i: (0, 0)),
            pl.BlockSpec((256, 8), lambda i: (0, 0)),
            pl.BlockSpec((1, 8), lambda i: (0, 0)),
        ],
        out_specs=pl.BlockSpec((RB_MLP, 8), lambda i: (i, 0)),
        out_shape=jax.ShapeDtypeStruct((N, 8), jnp.float32),
        interpret=interpret,
    )(f1, f2, f3, w7[:64], w7[64:128], w7[128:], b7[None, :], w8p, b8p)
    return out[:, :3]


def kernel(points, w1, b1, w2, b2, w3, b3, w4, b4, w5, b5, w6, b6,
           w7, b7, w8, b8):
    pts = points[0]
    nbr = _knn(pts)
    idx_flat = nbr.T.reshape(-1)

    pts16 = jnp.pad(pts, ((0, 0), (0, 13)))
    wc1 = jnp.pad(w1[:3] - w1[3:], ((0, 13), (0, 0)))
    wn1 = jnp.pad(w1[3:], ((0, 13), (0, 0)))
    xj1 = _sc_gather(pts16, idx_flat).reshape(KNN, N, 16)
    f1 = _edge_conv(pts16, xj1, wc1, wn1, b1[None, :], w2, b2[None, :])

    xj2 = _sc_gather(f1, idx_flat).reshape(KNN, N, 64)
    f2 = _edge_conv(f1, xj2, w3[:64] - w3[64:], w3[64:],
                    b3[None, :], w4, b4[None, :])

    xj3 = _sc_gather(f2, idx_flat).reshape(KNN, N, 64)
    f3 = _edge_conv(f2, xj3, w5[:64] - w5[64:], w5[64:],
                    b5[None, :], w6, b6[None, :])

    residual = _final_mlp(f1, f2, f3, w7, b7, w8, b8)
    return residual[None, :, :]


# speedup vs baseline: 1.0972x; 1.0972x over previous
import functools

import jax
import jax.numpy as jnp
from jax import lax
from jax.experimental import pallas as pl
from jax.experimental.pallas import tpu as pltpu
from jax.experimental.pallas import tpu_sc as plsc

N = 10000
KNN = 16
NPAD = 10240
SLAB = 256
NSLAB = NPAD // SLAB
SBITS = 6
SMASK = (1 << SBITS) - 1
DEPTH = 4
RB_KNN = 80
RB_CONV = 400
RB_MLP = 2000


def _knn_body(prows_ref, pcols_ref, out_ref):
    i = pl.program_id(0)
    xr = prows_ref[...]
    xc = pcols_ref[...]
    sq_r = jnp.sum(xr * xr, axis=1, keepdims=True)
    sq_c = jnp.sum(xc * xc, axis=0, keepdims=True)
    prod = jnp.dot(xr, xc, preferred_element_type=jnp.float32)
    d = sq_r + sq_c - 2.0 * prod
    inf = jnp.float32(jnp.inf)
    big = jnp.int32(NPAD)
    bias = jnp.int32(0x08000000)
    sent = jnp.int32(0x7F000000)
    sentf = jnp.float32(2.0 ** 127)
    row_ids = i * RB_KNN + lax.broadcasted_iota(jnp.int32, (RB_KNN, 1), 0)
    lane = lax.broadcasted_iota(jnp.int32, (RB_KNN, SLAB), 1)

    def _ce(a, b):
        return jnp.minimum(a, b), jnp.maximum(a, b)

    K = [jnp.full((RB_KNN, SLAB), sent) for _ in range(DEPTH)]
    grp = []
    for c in range(NSLAB):
        v = jnp.maximum(d[:, c * SLAB:(c + 1) * SLAB], 0.0)
        bits = lax.bitcast_convert_type(v, jnp.int32)
        vk = ((bits & jnp.int32(~SMASK)) | jnp.int32(c)) + bias
        if (c + 1) * SLAB > N:
            nv = max(0, N - c * SLAB)
            vk = jnp.where(lane >= jnp.int32(nv), sent, vk)
        grp.append(vk)
        if len(grp) == 4:
            a0, a1 = _ce(grp[0], grp[1])
            b0, b1 = _ce(grp[2], grp[3])
            g0, t1 = _ce(a0, b0)
            t2, g3 = _ce(a1, b1)
            g1, g2 = _ce(t1, t2)
            g = [g0, g1, g2, g3]
            L = [jnp.minimum(K[k], g[3 - k]) for k in range(4)]
            p0, p2 = _ce(L[0], L[2])
            p1, p3 = _ce(L[1], L[3])
            K0, K1 = _ce(p0, p1)
            K2, K3 = _ce(p2, p3)
            K = [K0, K1, K2, K3]
            grp = []

    F = [lax.bitcast_convert_type(k, jnp.float32) for k in K]
    cols = []
    qb15 = qb16 = None
    for t in range(KNN + 1):
        m = jnp.min(F[0], axis=1, keepdims=True)
        sel_lane = jnp.min(jnp.where(F[0] == m, lane, jnp.int32(SLAB)),
                           axis=1, keepdims=True)
        islane = lane == sel_lane
        mi = lax.bitcast_convert_type(m, jnp.int32) - bias
        cols.append((mi & jnp.int32(SMASK)) * SLAB + sel_lane)
        for k in range(DEPTH - 1):
            F[k] = jnp.where(islane, F[k + 1], F[k])
        F[DEPTH - 1] = jnp.where(islane, sentf, F[DEPTH - 1])
        if t == KNN - 1:
            qb15 = mi >> SBITS
        if t == KNN:
            qb16 = mi >> SBITS
    k18 = jnp.min(F[0], axis=1, keepdims=True)
    qb17 = (lax.bitcast_convert_type(k18, jnp.int32) - bias) >> SBITS
    drained = jnp.max(F[0], axis=1, keepdims=True) == sentf

    run = jnp.zeros((RB_KNN, 1), jnp.bool_)
    for t in range(KNN):
        run = run | (cols[t] == row_ids)
        out_ref[:, t:t + 1] = jnp.where(run, cols[t + 1], cols[t])
    eq16 = cols[KNN] == row_ids

    one = jnp.int32(1)
    zero = jnp.int32(0)
    c_a = jnp.where(qb17 <= qb16, one, zero)
    c_b = jnp.where(qb17 <= qb15, one, zero)
    c_c = jnp.where(qb16 <= qb15, one, zero)
    c_d = jnp.where(drained, one, zero)
    bad_i = jnp.maximum(jnp.where(run, c_a, jnp.where(eq16, c_b, c_c)), c_d)
    bad = jnp.max(bad_i) > 0

    @pl.when(bad)
    def _slow():
        col = lax.broadcasted_iota(jnp.int32, (RB_KNN, NPAD), 1)
        dd = jnp.where((col == row_ids) | (col >= N), inf, d)
        for t in range(KNN):
            mm = jnp.min(dd, axis=1, keepdims=True)
            cand = jnp.where(dd == mm, col, big)
            sel = jnp.min(cand, axis=1, keepdims=True)
            out_ref[:, t:t + 1] = sel
            dd = jnp.where(col == sel, inf, dd)


def _knn(pts, interpret=False):
    prows = jnp.pad(pts, ((0, 0), (0, 5)))
    pcols = jnp.pad(pts.T, ((0, 5), (0, NPAD - N)))
    return pl.pallas_call(
        _knn_body,
        grid=(N // RB_KNN,),
        in_specs=[
            pl.BlockSpec((RB_KNN, 8), lambda i: (i, 0)),
            pl.BlockSpec((8, NPAD), lambda i: (0, 0)),
        ],
        out_specs=pl.BlockSpec((RB_KNN, KNN), lambda i: (i, 0)),
        out_shape=jax.ShapeDtypeStruct((N, KNN), jnp.int32),
        interpret=interpret,
    )(prows, pcols)


def _sc_gather(table, idx_flat):
    E = idx_flat.shape[0]
    C = table.shape[1]
    info = plsc.get_sparse_core_info()
    nw = info.num_cores * info.num_subcores
    per_w = E // nw
    nch = per_w // 128
    rem = per_w - nch * 128
    mesh = plsc.VectorSubcoreMesh(core_axis_name="c", subcore_axis_name="s")

    @functools.partial(
        pl.kernel,
        mesh=mesh,
        compiler_params=pltpu.CompilerParams(use_tc_tiling_on_sc=False),
        out_type=jax.ShapeDtypeStruct((E, C), jnp.float32),
        scratch_types=[
            pltpu.VMEM((128,), jnp.int32),
            pltpu.VMEM((128, C), jnp.float32),
            pltpu.VMEM((8,), jnp.int32),
            pltpu.VMEM((8, C), jnp.float32),
            pltpu.SemaphoreType.DMA,
        ],
    )
    def gk(table_hbm, idx_hbm, out_hbm, idx_v, rows_v, idx_v2, rows_v2, sem):
        wid = lax.axis_index("s") * info.num_cores + lax.axis_index("c")
        base = wid * per_w

        def body(ci, carry):
            off = base + ci * 128
            pltpu.sync_copy(idx_hbm.at[pl.ds(off, 128)], idx_v)
            pltpu.async_copy(table_hbm.at[idx_v], rows_v, sem).wait()
            pltpu.sync_copy(rows_v, out_hbm.at[pl.ds(off, 128)])
            return carry

        lax.fori_loop(0, nch, body, 0)
        if rem:
            off2 = base + nch * 128
            pltpu.sync_copy(idx_hbm.at[pl.ds(off2, rem)], idx_v2)
            pltpu.async_copy(table_hbm.at[idx_v2], rows_v2, sem).wait()
            pltpu.sync_copy(rows_v2, out_hbm.at[pl.ds(off2, rem)])

    return gk(table, idx_flat)


def _conv_body(x_ref, xj_ref, wc_ref, wn_ref, b1_ref, w2_ref, b2_ref, out_ref):
    x = x_ref[...]
    tcv = jnp.dot(x, wc_ref[...], preferred_element_type=jnp.float32) \
        + b1_ref[...]
    acc = None
    for j in range(KNN):
        xj = xj_ref[j]
        pre = jnp.dot(xj, wn_ref[...], preferred_element_type=jnp.float32) + tcv
        hj = jnp.maximum(pre, 0.0)
        oj = jnp.dot(hj, w2_ref[...], preferred_element_type=jnp.float32)
        acc = oj if acc is None else jnp.maximum(acc, oj)
    out_ref[...] = acc + b2_ref[...]


def _edge_conv(x, xj3, wc, wn, b1, w2, b2, interpret=False):
    C = x.shape[1]
    return pl.pallas_call(
        _conv_body,
        grid=(N // RB_CONV,),
        in_specs=[
            pl.BlockSpec((RB_CONV, C), lambda i: (i, 0)),
            pl.BlockSpec((KNN, RB_CONV, C), lambda i: (0, i, 0)),
            pl.BlockSpec((C, 64), lambda i: (0, 0)),
            pl.BlockSpec((C, 64), lambda i: (0, 0)),
            pl.BlockSpec((1, 64), lambda i: (0, 0)),
            pl.BlockSpec((64, 64), lambda i: (0, 0)),
            pl.BlockSpec((1, 64), lambda i: (0, 0)),
        ],
        out_specs=pl.BlockSpec((RB_CONV, 64), lambda i: (i, 0)),
        out_shape=jax.ShapeDtypeStruct((N, 64), jnp.float32),
        interpret=interpret,
    )(x, xj3, wc, wn, b1, w2, b2)


def _mlp_body(f1_ref, f2_ref, f3_ref, wa_ref, wb_ref, wc_ref, b7_ref,
              w8_ref, b8_ref, out_ref):
    t = (jnp.dot(f1_ref[...], wa_ref[...], preferred_element_type=jnp.float32)
         + jnp.dot(f2_ref[...], wb_ref[...], preferred_element_type=jnp.float32)
         + jnp.dot(f3_ref[...], wc_ref[...], preferred_element_type=jnp.float32)
         + b7_ref[...])
    t = jnp.maximum(t, 0.0)
    out_ref[...] = jnp.dot(t, w8_ref[...], preferred_element_type=jnp.float32) \
        + b8_ref[...]


def _final_mlp(f1, f2, f3, w7, b7, w8, b8, interpret=False):
    w8p = jnp.pad(w8, ((0, 0), (0, 8 - w8.shape[1])))
    b8p = jnp.pad(b8, (0, 8 - b8.shape[0]))[None, :]
    out = pl.pallas_call(
        _mlp_body,
        grid=(N // RB_MLP,),
        in_specs=[
            pl.BlockSpec((RB_MLP, 64), lambda i: (i, 0)),
            pl.BlockSpec((RB_MLP, 64), lambda i: (i, 0)),
            pl.BlockSpec((RB_MLP, 64), lambda i: (i, 0)),
            pl.BlockSpec((64, 256), lambda i: (0, 0)),
            pl.BlockSpec((64, 256), lambda i: (0, 0)),
            pl.BlockSpec((64, 256), lambda i: (0, 0)),
            pl.BlockSpec((1, 256), lambda i: (0, 0)),
            pl.BlockSpec((256, 8), lambda i: (0, 0)),
            pl.BlockSpec((1, 8), lambda i: (0, 0)),
        ],
        out_specs=pl.BlockSpec((RB_MLP, 8), lambda i: (i, 0)),
        out_shape=jax.ShapeDtypeStruct((N, 8), jnp.float32),
        interpret=interpret,
    )(f1, f2, f3, w7[:64], w7[64:128], w7[128:], b7[None, :], w8p, b8p)
    return out[:, :3]


def kernel(points, w1, b1, w2, b2, w3, b3, w4, b4, w5, b5, w6, b6,
           w7, b7, w8, b8):
    pts = points[0]
    nbr = _knn(pts)
    idx_flat = nbr.T.reshape(-1)

    pts16 = jnp.pad(pts, ((0, 0), (0, 13)))
    wc1 = jnp.pad(w1[:3] - w1[3:], ((0, 13), (0, 0)))
    wn1 = jnp.pad(w1[3:], ((0, 13), (0, 0)))
    xj1 = _sc_gather(pts16, idx_flat).reshape(KNN, N, 16)
    f1 = _edge_conv(pts16, xj1, wc1, wn1, b1[None, :], w2, b2[None, :])

    xj2 = _sc_gather(f1, idx_flat).reshape(KNN, N, 64)
    f2 = _edge_conv(f1, xj2, w3[:64] - w3[64:], w3[64:],
                    b3[None, :], w4, b4[None, :])

    xj3 = _sc_gather(f2, idx_flat).reshape(KNN, N, 64)
    f3 = _edge_conv(f2, xj3, w5[:64] - w5[64:], w5[64:],
                    b5[None, :], w6, b6[None, :])

    residual = _final_mlp(f1, f2, f3, w7, b7, w8, b8)
    return residual[None, :, :]


# speedup vs baseline: 1.2508x; 1.1400x over previous
import functools

import jax
import jax.numpy as jnp
from jax import lax
from jax.experimental import pallas as pl
from jax.experimental.pallas import tpu as pltpu
from jax.experimental.pallas import tpu_sc as plsc

N = 10000
KNN = 16
NPAD = 10240
SLAB = 256
NSLAB = NPAD // SLAB
SBITS = 6
SMASK = (1 << SBITS) - 1
DEPTH = 4
RB_KNN = 200
RB_CONV = 400
RB_MLP = 2000


def _knn_body(prows_ref, pcols_ref, out_ref):
    i = pl.program_id(0)
    xr = prows_ref[...]
    xc = pcols_ref[...]
    sq_r = jnp.sum(xr * xr, axis=1, keepdims=True)
    sq_c = jnp.sum(xc * xc, axis=0, keepdims=True)
    prod = jnp.dot(xr, xc, preferred_element_type=jnp.float32)
    d = sq_r + sq_c - 2.0 * prod
    inf = jnp.float32(jnp.inf)
    big = jnp.int32(NPAD)
    bias = jnp.int32(0x08000000)
    sent = jnp.int32(0x7F000000)
    sentf = jnp.float32(2.0 ** 127)
    row_ids = i * RB_KNN + lax.broadcasted_iota(jnp.int32, (RB_KNN, 1), 0)
    lane = lax.broadcasted_iota(jnp.int32, (RB_KNN, SLAB), 1)

    def _ce(a, b):
        return jnp.minimum(a, b), jnp.maximum(a, b)

    K = [jnp.full((RB_KNN, SLAB), sent) for _ in range(DEPTH)]
    grp = []
    for c in range(NSLAB):
        v = jnp.maximum(d[:, c * SLAB:(c + 1) * SLAB], 0.0)
        bits = lax.bitcast_convert_type(v, jnp.int32)
        vk = ((bits & jnp.int32(~SMASK)) | jnp.int32(c)) + bias
        if (c + 1) * SLAB > N:
            nv = max(0, N - c * SLAB)
            vk = jnp.where(lane >= jnp.int32(nv), sent, vk)
        grp.append(vk)
        if len(grp) == 4:
            a0, a1 = _ce(grp[0], grp[1])
            b0, b1 = _ce(grp[2], grp[3])
            g0, t1 = _ce(a0, b0)
            t2, g3 = _ce(a1, b1)
            g1, g2 = _ce(t1, t2)
            g = [g0, g1, g2, g3]
            L = [jnp.minimum(K[k], g[3 - k]) for k in range(4)]
            p0, p2 = _ce(L[0], L[2])
            p1, p3 = _ce(L[1], L[3])
            K0, K1 = _ce(p0, p1)
            K2, K3 = _ce(p2, p3)
            K = [K0, K1, K2, K3]
            grp = []

    F = [lax.bitcast_convert_type(k, jnp.float32) for k in K]
    cols = []
    qb15 = qb16 = None
    for t in range(KNN + 1):
        m = jnp.min(F[0], axis=1, keepdims=True)
        sel_lane = jnp.min(jnp.where(F[0] == m, lane, jnp.int32(SLAB)),
                           axis=1, keepdims=True)
        islane = lane == sel_lane
        mi = lax.bitcast_convert_type(m, jnp.int32) - bias
        cols.append((mi & jnp.int32(SMASK)) * SLAB + sel_lane)
        for k in range(DEPTH - 1):
            F[k] = jnp.where(islane, F[k + 1], F[k])
        F[DEPTH - 1] = jnp.where(islane, sentf, F[DEPTH - 1])
        if t == KNN - 1:
            qb15 = mi >> SBITS
        if t == KNN:
            qb16 = mi >> SBITS
    k18 = jnp.min(F[0], axis=1, keepdims=True)
    qb17 = (lax.bitcast_convert_type(k18, jnp.int32) - bias) >> SBITS
    drained = jnp.max(F[0], axis=1, keepdims=True) == sentf

    run = jnp.zeros((RB_KNN, 1), jnp.bool_)
    for t in range(KNN):
        run = run | (cols[t] == row_ids)
        out_ref[:, t:t + 1] = jnp.where(run, cols[t + 1], cols[t])
    eq16 = cols[KNN] == row_ids

    one = jnp.int32(1)
    zero = jnp.int32(0)
    c_a = jnp.where(qb17 <= qb16, one, zero)
    c_b = jnp.where(qb17 <= qb15, one, zero)
    c_c = jnp.where(qb16 <= qb15, one, zero)
    c_d = jnp.where(drained, one, zero)
    bad_i = jnp.maximum(jnp.where(run, c_a, jnp.where(eq16, c_b, c_c)), c_d)
    bad = jnp.max(bad_i) > 0

    @pl.when(bad)
    def _slow():
        col = lax.broadcasted_iota(jnp.int32, (RB_KNN, NPAD), 1)
        dd = jnp.where((col == row_ids) | (col >= N), inf, d)
        for t in range(KNN):
            mm = jnp.min(dd, axis=1, keepdims=True)
            cand = jnp.where(dd == mm, col, big)
            sel = jnp.min(cand, axis=1, keepdims=True)
            out_ref[:, t:t + 1] = sel
            dd = jnp.where(col == sel, inf, dd)


def _knn(pts, interpret=False):
    prows = jnp.pad(pts, ((0, 0), (0, 5)))
    pcols = jnp.pad(pts.T, ((0, 5), (0, NPAD - N)))
    return pl.pallas_call(
        _knn_body,
        grid=(N // RB_KNN,),
        in_specs=[
            pl.BlockSpec((RB_KNN, 8), lambda i: (i, 0)),
            pl.BlockSpec((8, NPAD), lambda i: (0, 0)),
        ],
        out_specs=pl.BlockSpec((RB_KNN, KNN), lambda i: (i, 0)),
        out_shape=jax.ShapeDtypeStruct((N, KNN), jnp.int32),
        interpret=interpret,
    )(prows, pcols)


def _sc_gather(table, idx_flat):
    E = idx_flat.shape[0]
    C = table.shape[1]
    info = plsc.get_sparse_core_info()
    nw = info.num_cores * info.num_subcores
    per_w = E // nw
    nch = per_w // 128
    rem = per_w - nch * 128
    mesh = plsc.VectorSubcoreMesh(core_axis_name="c", subcore_axis_name="s")

    @functools.partial(
        pl.kernel,
        mesh=mesh,
        compiler_params=pltpu.CompilerParams(use_tc_tiling_on_sc=False),
        out_type=jax.ShapeDtypeStruct((E, C), jnp.float32),
        scratch_types=[
            pltpu.VMEM((128,), jnp.int32),
            pltpu.VMEM((128, C), jnp.float32),
            pltpu.VMEM((8,), jnp.int32),
            pltpu.VMEM((8, C), jnp.float32),
            pltpu.SemaphoreType.DMA,
        ],
    )
    def gk(table_hbm, idx_hbm, out_hbm, idx_v, rows_v, idx_v2, rows_v2, sem):
        wid = lax.axis_index("s") * info.num_cores + lax.axis_index("c")
        base = wid * per_w

        def body(ci, carry):
            off = base + ci * 128
            pltpu.sync_copy(idx_hbm.at[pl.ds(off, 128)], idx_v)
            pltpu.async_copy(table_hbm.at[idx_v], rows_v, sem).wait()
            pltpu.sync_copy(rows_v, out_hbm.at[pl.ds(off, 128)])
            return carry

        lax.fori_loop(0, nch, body, 0)
        if rem:
            off2 = base + nch * 128
            pltpu.sync_copy(idx_hbm.at[pl.ds(off2, rem)], idx_v2)
            pltpu.async_copy(table_hbm.at[idx_v2], rows_v2, sem).wait()
            pltpu.sync_copy(rows_v2, out_hbm.at[pl.ds(off2, rem)])

    return gk(table, idx_flat)


def _conv_body(x_ref, xj_ref, wc_ref, wn_ref, b1_ref, w2_ref, b2_ref, out_ref):
    x = x_ref[...]
    tcv = jnp.dot(x, wc_ref[...], preferred_element_type=jnp.float32) \
        + b1_ref[...]
    acc = None
    for j in range(KNN):
        xj = xj_ref[j]
        pre = jnp.dot(xj, wn_ref[...], preferred_element_type=jnp.float32) + tcv
        hj = jnp.maximum(pre, 0.0)
        oj = jnp.dot(hj, w2_ref[...], preferred_element_type=jnp.float32)
        acc = oj if acc is None else jnp.maximum(acc, oj)
    out_ref[...] = acc + b2_ref[...]


def _edge_conv(x, xj3, wc, wn, b1, w2, b2, interpret=False):
    C = x.shape[1]
    return pl.pallas_call(
        _conv_body,
        grid=(N // RB_CONV,),
        in_specs=[
            pl.BlockSpec((RB_CONV, C), lambda i: (i, 0)),
            pl.BlockSpec((KNN, RB_CONV, C), lambda i: (0, i, 0)),
            pl.BlockSpec((C, 64), lambda i: (0, 0)),
            pl.BlockSpec((C, 64), lambda i: (0, 0)),
            pl.BlockSpec((1, 64), lambda i: (0, 0)),
            pl.BlockSpec((64, 64), lambda i: (0, 0)),
            pl.BlockSpec((1, 64), lambda i: (0, 0)),
        ],
        out_specs=pl.BlockSpec((RB_CONV, 64), lambda i: (i, 0)),
        out_shape=jax.ShapeDtypeStruct((N, 64), jnp.float32),
        interpret=interpret,
    )(x, xj3, wc, wn, b1, w2, b2)


def _mlp_body(f1_ref, f2_ref, f3_ref, wa_ref, wb_ref, wc_ref, b7_ref,
              w8_ref, b8_ref, out_ref):
    t = (jnp.dot(f1_ref[...], wa_ref[...], preferred_element_type=jnp.float32)
         + jnp.dot(f2_ref[...], wb_ref[...], preferred_element_type=jnp.float32)
         + jnp.dot(f3_ref[...], wc_ref[...], preferred_element_type=jnp.float32)
         + b7_ref[...])
    t = jnp.maximum(t, 0.0)
    out_ref[...] = jnp.dot(t, w8_ref[...], preferred_element_type=jnp.float32) \
        + b8_ref[...]


def _final_mlp(f1, f2, f3, w7, b7, w8, b8, interpret=False):
    w8p = jnp.pad(w8, ((0, 0), (0, 8 - w8.shape[1])))
    b8p = jnp.pad(b8, (0, 8 - b8.shape[0]))[None, :]
    out = pl.pallas_call(
        _mlp_body,
        grid=(N // RB_MLP,),
        in_specs=[
            pl.BlockSpec((RB_MLP, 64), lambda i: (i, 0)),
            pl.BlockSpec((RB_MLP, 64), lambda i: (i, 0)),
            pl.BlockSpec((RB_MLP, 64), lambda i: (i, 0)),
            pl.BlockSpec((64, 256), lambda i: (0, 0)),
            pl.BlockSpec((64, 256), lambda i: (0, 0)),
            pl.BlockSpec((64, 256), lambda i: (0, 0)),
            pl.BlockSpec((1, 256), lambda i: (0, 0)),
            pl.BlockSpec((256, 8), lambda i: (0, 0)),
            pl.BlockSpec((1, 8), lambda i: (0, 0)),
        ],
        out_specs=pl.BlockSpec((RB_MLP, 8), lambda i: (i, 0)),
        out_shape=jax.ShapeDtypeStruct((N, 8), jnp.float32),
        interpret=interpret,
    )(f1, f2, f3, w7[:64], w7[64:128], w7[128:], b7[None, :], w8p, b8p)
    return out[:, :3]


def kernel(points, w1, b1, w2, b2, w3, b3, w4, b4, w5, b5, w6, b6,
           w7, b7, w8, b8):
    pts = points[0]
    nbr = _knn(pts)
    idx_flat = nbr.T.reshape(-1)

    pts16 = jnp.pad(pts, ((0, 0), (0, 13)))
    wc1 = jnp.pad(w1[:3] - w1[3:], ((0, 13), (0, 0)))
    wn1 = jnp.pad(w1[3:], ((0, 13), (0, 0)))
    xj1 = _sc_gather(pts16, idx_flat).reshape(KNN, N, 16)
    f1 = _edge_conv(pts16, xj1, wc1, wn1, b1[None, :], w2, b2[None, :])

    xj2 = _sc_gather(f1, idx_flat).reshape(KNN, N, 64)
    f2 = _edge_conv(f1, xj2, w3[:64] - w3[64:], w3[64:],
                    b3[None, :], w4, b4[None, :])

    xj3 = _sc_gather(f2, idx_flat).reshape(KNN, N, 64)
    f3 = _edge_conv(f2, xj3, w5[:64] - w5[64:], w5[64:],
                    b5[None, :], w6, b6[None, :])

    residual = _final_mlp(f1, f2, f3, w7, b7, w8, b8)
    return residual[None, :, :]
